# baseline (speedup 1.0000x reference)
import functools

import jax
import jax.numpy as jnp
from jax.experimental import pallas as pl

D_MODEL = 1024
D_FF = 4096
E = 8
RANK = 4
ER = E * RANK
TM = 512
FC = 1024

_DN_T = (((1,), (1,)), ((), ()))


def _moe_tile(x_ref, gate_w_ref, gate_b_ref, a_all_ref, b_cat_ref,
              wi_w_ref, wi_b_ref, wo_w_ref, wo_b_ref, out_ref):
    x = x_ref[...]
    xb = x.astype(jnp.bfloat16)

    logits = jax.lax.dot_general(
        x, gate_w_ref[...], _DN_T,
        preferred_element_type=jnp.float32) + gate_b_ref[...]
    m = jnp.max(logits, axis=-1, keepdims=True)
    idx = jax.lax.broadcasted_iota(jnp.int32, logits.shape, 1)
    e_sel = jnp.min(jnp.where(logits >= m, idx, E), axis=-1, keepdims=True)

    a = jax.lax.dot_general(
        x, a_all_ref[...], _DN_T,
        preferred_element_type=jnp.float32)
    col_expert = jax.lax.broadcasted_iota(jnp.int32, a.shape, 1) // RANK
    a_masked = jnp.where(col_expert == e_sel, a, 0.0).astype(jnp.bfloat16)

    acc = jnp.zeros((x.shape[0], D_MODEL), jnp.float32)
    for c in range(D_FF // FC):
        sl = pl.ds(c * FC, FC)
        base = jax.lax.dot_general(
            xb, wi_w_ref[sl, :], _DN_T,
            preferred_element_type=jnp.float32)
        lora = jax.lax.dot_general(
            a_masked, b_cat_ref[sl, :], _DN_T,
            preferred_element_type=jnp.float32)
        inter = jnp.maximum(base + lora + wi_b_ref[:, sl], 0.0)
        acc = acc + jax.lax.dot_general(
            inter.astype(jnp.bfloat16), wo_w_ref[:, sl], _DN_T,
            preferred_element_type=jnp.float32)
    out_ref[...] = acc + wo_b_ref[...]


@functools.partial(jax.jit, static_argnames=("interpret",))
def _moe_forward(x, gate_w, gate_b, a_all, b_cat, wi_w, wi_b, wo_w, wo_b,
                 interpret=False):
    t = x.shape[0]
    grid = (t // TM,)
    full = lambda shape: pl.BlockSpec(shape, lambda i: (0,) * len(shape))
    return pl.pallas_call(
        _moe_tile,
        grid=grid,
        in_specs=[
            pl.BlockSpec((TM, D_MODEL), lambda i: (i, 0)),
            full((E, D_MODEL)),
            full((1, E)),
            full((ER, D_MODEL)),
            full((D_FF, ER)),
            full((D_FF, D_MODEL)),
            full((1, D_FF)),
            full((D_MODEL, D_FF)),
            full((1, D_MODEL)),
        ],
        out_specs=pl.BlockSpec((TM, D_MODEL), lambda i: (i, 0)),
        out_shape=jax.ShapeDtypeStruct((t, D_MODEL), jnp.float32),
        interpret=interpret,
    )(x, gate_w, gate_b, a_all, b_cat, wi_w, wi_b, wo_w, wo_b)


def kernel(hidden_states, gate_w, gate_b, wi_w, wi_b, wo_w, wo_b,
           lora_A, lora_B, interpret=False):
    b, s, d = hidden_states.shape
    x = hidden_states.reshape(b * s, d)
    a_all = lora_A.reshape(ER, D_MODEL)
    b_cat = jnp.transpose(lora_B, (1, 0, 2)).reshape(D_FF, ER).astype(jnp.bfloat16)
    out = _moe_forward(x, gate_w, gate_b.reshape(1, E), a_all, b_cat,
                       wi_w.astype(jnp.bfloat16), wi_b.reshape(1, D_FF),
                       wo_w.astype(jnp.bfloat16),
                       wo_b.reshape(1, D_MODEL), interpret=interpret)
    return out.reshape(b, s, d)


# speedup vs baseline: 1.0845x; 1.0845x over previous
import functools

import jax
import jax.numpy as jnp
from jax.experimental import pallas as pl

D_MODEL = 1024
D_FF = 4096
E = 8
RANK = 4
ER = E * RANK
TM = 512
FC = 1024

_DN_T = (((1,), (1,)), ((), ()))


def _moe_tile(x_ref, gate_w_ref, gate_b_ref, a_all_ref, b_cat_ref,
              wi_w_ref, wi_b_ref, wo_w_ref, wo_b_ref, out_ref):
    x = x_ref[...]

    logits = jax.lax.dot_general(
        x, gate_w_ref[...], _DN_T,
        preferred_element_type=jnp.float32) + gate_b_ref[...]
    m = jnp.max(logits, axis=-1, keepdims=True)
    idx = jax.lax.broadcasted_iota(jnp.int32, logits.shape, 1)
    e_sel = jnp.min(jnp.where(logits >= m, idx, E), axis=-1, keepdims=True)

    a = jax.lax.dot_general(
        x, a_all_ref[...], _DN_T,
        preferred_element_type=jnp.float32)
    col_expert = jax.lax.broadcasted_iota(jnp.int32, a.shape, 1) // RANK
    a_masked = jnp.where(col_expert == e_sel, a, 0.0)
    xa = jnp.concatenate([x, a_masked], axis=1)

    acc = jnp.zeros((x.shape[0], D_MODEL), jnp.float32)
    for c in range(D_FF // FC):
        sl = pl.ds(c * FC, FC)
        w_cat = jnp.concatenate(
            [wi_w_ref[sl, :], b_cat_ref[sl, :]], axis=1)
        inter = jnp.maximum(
            jax.lax.dot_general(
                xa, w_cat, _DN_T, preferred_element_type=jnp.float32)
            + wi_b_ref[:, sl], 0.0)
        acc = acc + jax.lax.dot_general(
            inter, wo_w_ref[:, sl], _DN_T,
            preferred_element_type=jnp.float32)
    out_ref[...] = acc + wo_b_ref[...]


@functools.partial(jax.jit, static_argnames=("interpret",))
def _moe_forward(x, gate_w, gate_b, a_all, b_cat, wi_w, wi_b, wo_w, wo_b,
                 interpret=False):
    t = x.shape[0]
    grid = (t // TM,)
    full = lambda shape: pl.BlockSpec(shape, lambda i: (0,) * len(shape))
    return pl.pallas_call(
        _moe_tile,
        grid=grid,
        in_specs=[
            pl.BlockSpec((TM, D_MODEL), lambda i: (i, 0)),
            full((E, D_MODEL)),
            full((1, E)),
            full((ER, D_MODEL)),
            full((D_FF, ER)),
            full((D_FF, D_MODEL)),
            full((1, D_FF)),
            full((D_MODEL, D_FF)),
            full((1, D_MODEL)),
        ],
        out_specs=pl.BlockSpec((TM, D_MODEL), lambda i: (i, 0)),
        out_shape=jax.ShapeDtypeStruct((t, D_MODEL), jnp.float32),
        interpret=interpret,
    )(x, gate_w, gate_b, a_all, b_cat, wi_w, wi_b, wo_w, wo_b)


def kernel(hidden_states, gate_w, gate_b, wi_w, wi_b, wo_w, wo_b,
           lora_A, lora_B, interpret=False):
    b, s, d = hidden_states.shape
    x = hidden_states.reshape(b * s, d)
    a_all = lora_A.reshape(ER, D_MODEL)
    b_cat = jnp.transpose(lora_B, (1, 0, 2)).reshape(D_FF, ER)
    out = _moe_forward(x, gate_w, gate_b.reshape(1, E), a_all, b_cat,
                       wi_w, wi_b.reshape(1, D_FF), wo_w,
                       wo_b.reshape(1, D_MODEL), interpret=interpret)
    return out.reshape(b, s, d)


# speedup vs baseline: 1.0908x; 1.0058x over previous
import functools

import jax
import jax.numpy as jnp
from jax.experimental import pallas as pl

D_MODEL = 1024
D_FF = 4096
E = 8
RANK = 4
ER = E * RANK
PJ = E + ER
TM = 512
FC = 1024

_DN_T = (((1,), (1,)), ((), ()))


def _moe_tile(x_ref, ga_ref, gate_b_ref, bz_ref,
              wi_w_ref, wi_b_ref, wo_w_ref, wo_b_ref, out_ref):
    x = x_ref[...]

    proj = jax.lax.dot_general(
        x, ga_ref[...], _DN_T,
        preferred_element_type=jnp.float32) + gate_b_ref[...]

    col = jax.lax.broadcasted_iota(jnp.int32, proj.shape, 1)
    is_logit = col < E
    m = jnp.max(jnp.where(is_logit, proj, -jnp.inf), axis=-1, keepdims=True)
    e_sel = jnp.min(jnp.where(is_logit & (proj >= m), col, E),
                    axis=-1, keepdims=True)

    col_expert = (col - E) // RANK
    proj_masked = jnp.where(col_expert == e_sel, proj, 0.0)
    xa = jnp.concatenate([x, proj_masked], axis=1)

    acc = jnp.zeros((x.shape[0], D_MODEL), jnp.float32)
    for c in range(D_FF // FC):
        sl = pl.ds(c * FC, FC)
        w_cat = jnp.concatenate(
            [wi_w_ref[sl, :], bz_ref[sl, :]], axis=1)
        inter = jnp.maximum(
            jax.lax.dot_general(
                xa, w_cat, _DN_T, preferred_element_type=jnp.float32)
            + wi_b_ref[:, sl], 0.0)
        acc = acc + jax.lax.dot_general(
            inter, wo_w_ref[:, sl], _DN_T,
            preferred_element_type=jnp.float32)
    out_ref[...] = acc + wo_b_ref[...]


@functools.partial(jax.jit, static_argnames=("interpret",))
def _moe_forward(x, ga, gate_b, bz, wi_w, wi_b, wo_w, wo_b,
                 interpret=False):
    t = x.shape[0]
    grid = (t // TM,)
    full = lambda shape: pl.BlockSpec(shape, lambda i: (0,) * len(shape))
    return pl.pallas_call(
        _moe_tile,
        grid=grid,
        in_specs=[
            pl.BlockSpec((TM, D_MODEL), lambda i: (i, 0)),
            full((PJ, D_MODEL)),
            full((1, PJ)),
            full((D_FF, PJ)),
            full((D_FF, D_MODEL)),
            full((1, D_FF)),
            full((D_MODEL, D_FF)),
            full((1, D_MODEL)),
        ],
        out_specs=pl.BlockSpec((TM, D_MODEL), lambda i: (i, 0)),
        out_shape=jax.ShapeDtypeStruct((t, D_MODEL), jnp.float32),
        interpret=interpret,
    )(x, ga, gate_b, bz, wi_w, wi_b, wo_w, wo_b)


def kernel(hidden_states, gate_w, gate_b, wi_w, wi_b, wo_w, wo_b,
           lora_A, lora_B, interpret=False):
    b, s, d = hidden_states.shape
    x = hidden_states.reshape(b * s, d)
    ga = jnp.concatenate([gate_w, lora_A.reshape(ER, D_MODEL)], axis=0)
    gb = jnp.concatenate([gate_b, jnp.zeros((ER,), gate_b.dtype)])
    b_cat = jnp.transpose(lora_B, (1, 0, 2)).reshape(D_FF, ER)
    bz = jnp.concatenate([jnp.zeros((D_FF, E), b_cat.dtype), b_cat], axis=1)
    out = _moe_forward(x, ga, gb.reshape(1, PJ), bz,
                       wi_w, wi_b.reshape(1, D_FF), wo_w,
                       wo_b.reshape(1, D_MODEL), interpret=interpret)
    return out.reshape(b, s, d)


# speedup vs baseline: 1.0909x; 1.0001x over previous
import jax
import jax.numpy as jnp
from jax.experimental import pallas as pl

D_MODEL = 1024
D_FF = 4096
E = 8
RANK = 4
ER = E * RANK
PJ = E + ER
TM = 512
FC = 1024

_DN_T = (((1,), (1,)), ((), ()))


def _moe_tile(x_ref, ga_ref, gate_b_ref, bz_ref,
              wi_w_ref, wi_b_ref, wo_w_ref, wo_b_ref, out_ref):
    x = x_ref[...]

    proj = jax.lax.dot_general(
        x, ga_ref[...], _DN_T,
        preferred_element_type=jnp.float32) + gate_b_ref[...]

    col = jax.lax.broadcasted_iota(jnp.int32, proj.shape, 1)
    is_logit = col < E
    m = jnp.max(jnp.where(is_logit, proj, -jnp.inf), axis=-1, keepdims=True)
    e_sel = jnp.min(jnp.where(is_logit & (proj >= m), col, E),
                    axis=-1, keepdims=True)

    col_expert = (col - E) // RANK
    proj_masked = jnp.where(col_expert == e_sel, proj, 0.0)
    xa = jnp.concatenate([x, proj_masked], axis=1)

    acc = jnp.zeros((x.shape[0], D_MODEL), jnp.float32)
    for c in range(D_FF // FC):
        sl = pl.ds(c * FC, FC)
        w_cat = jnp.concatenate(
            [wi_w_ref[sl, :], bz_ref[sl, :]], axis=1)
        inter = jnp.maximum(
            jax.lax.dot_general(
                xa, w_cat, _DN_T, preferred_element_type=jnp.float32)
            + wi_b_ref[:, sl], 0.0)
        acc = acc + jax.lax.dot_general(
            inter, wo_w_ref[:, sl], _DN_T,
            preferred_element_type=jnp.float32)
    out_ref[...] = acc + wo_b_ref[...]


@jax.jit
def _moe_forward(x, ga, gate_b, bz, wi_w, wi_b, wo_w, wo_b):
    t = x.shape[0]
    grid = (t // TM,)
    full = lambda shape: pl.BlockSpec(shape, lambda i: (0,) * len(shape))
    return pl.pallas_call(
        _moe_tile,
        grid=grid,
        in_specs=[
            pl.BlockSpec((TM, D_MODEL), lambda i: (i, 0)),
            full((PJ, D_MODEL)),
            full((1, PJ)),
            full((D_FF, PJ)),
            full((D_FF, D_MODEL)),
            full((1, D_FF)),
            full((D_MODEL, D_FF)),
            full((1, D_MODEL)),
        ],
        out_specs=pl.BlockSpec((TM, D_MODEL), lambda i: (i, 0)),
        out_shape=jax.ShapeDtypeStruct((t, D_MODEL), jnp.float32),
    )(x, ga, gate_b, bz, wi_w, wi_b, wo_w, wo_b)


def kernel(hidden_states, gate_w, gate_b, wi_w, wi_b, wo_w, wo_b,
           lora_A, lora_B):
    b, s, d = hidden_states.shape
    x = hidden_states.reshape(b * s, d)
    ga = jnp.concatenate([gate_w, lora_A.reshape(ER, D_MODEL)], axis=0)
    gb = jnp.concatenate([gate_b, jnp.zeros((ER,), gate_b.dtype)])
    b_cat = jnp.transpose(lora_B, (1, 0, 2)).reshape(D_FF, ER)
    bz = jnp.concatenate([jnp.zeros((D_FF, E), b_cat.dtype), b_cat], axis=1)
    out = _moe_forward(x, ga, gb.reshape(1, PJ), bz,
                       wi_w, wi_b.reshape(1, D_FF), wo_w,
                       wo_b.reshape(1, D_MODEL))
    return out.reshape(b, s, d)
